# baseline (speedup 1.0000x reference)
import dataclasses
import functools

import jax
import jax.numpy as jnp
from jax import lax
from jax.experimental import pallas as pl
from jax.experimental.pallas import tpu as pltpu
from jax.experimental.pallas import tpu_sc as plsc

BATCH = 16384
HIDDEN = 32
NUM_CORES = 2
NUM_SUBCORES = 16
NUM_WORKERS = NUM_CORES * NUM_SUBCORES
B_PER_W = BATCH // NUM_WORKERS
LANES = 16
GROUPS = B_PER_W // LANES
IDX_CHUNK = 128
N_CHUNKS = B_PER_W // IDX_CHUNK


def _mf_kernel(uid_hbm, iid_hbm, avg_hbm, uembT_hbm, iembT_hbm, ubias_hbm,
               ibias_hbm, out_hbm, uid_v, iid_v, avg_v, gu, gi,
               ub_v, ib_v, out_v, sem0, sem1, sem2, sem3):
  wid = lax.axis_index("s") * NUM_CORES + lax.axis_index("c")
  base = wid * B_PER_W

  for k in range(N_CHUNKS):
    pltpu.sync_copy(uid_hbm.at[pl.ds(base + k * IDX_CHUNK, IDX_CHUNK)],
                    uid_v.at[k])
    pltpu.sync_copy(iid_hbm.at[pl.ds(base + k * IDX_CHUNK, IDX_CHUNK)],
                    iid_v.at[k])
  pltpu.sync_copy(avg_hbm.at[pl.ds(base, B_PER_W)], avg_v)

  copies = []
  for k in range(N_CHUNKS):
    lo = k * IDX_CHUNK
    copies.append(pltpu.async_copy(
        ubias_hbm.at[uid_v.at[k]], ub_v.at[pl.ds(lo, IDX_CHUNK)], sem2))
    copies.append(pltpu.async_copy(
        ibias_hbm.at[iid_v.at[k]], ib_v.at[pl.ds(lo, IDX_CHUNK)], sem3))

  gcopies = []
  for h in range(HIDDEN):
    for q in range(N_CHUNKS):
      r = h * N_CHUNKS + q
      gcopies.append(pltpu.async_copy(
          uembT_hbm.at[h].at[uid_v.at[q]], gu.at[r], sem0))
      gcopies.append(pltpu.async_copy(
          iembT_hbm.at[h].at[iid_v.at[q]], gi.at[r], sem1))
  for c in gcopies:
    c.wait()
  for c in copies:
    c.wait()

  @pl.loop(0, GROUPS)
  def _(g):
    q = g // (GROUPS // N_CHUNKS)
    j0 = (g % (GROUPS // N_CHUNKS)) * LANES
    b0 = g * LANES
    acc = avg_v[pl.ds(b0, LANES)] + ub_v[pl.ds(b0, LANES)] + ib_v[pl.ds(b0, LANES)]
    for h in range(HIDDEN):
      r = h * N_CHUNKS + q
      acc = acc + gu[r, pl.ds(j0, LANES)] * gi[r, pl.ds(j0, LANES)]
    out_v[pl.ds(b0, LANES)] = acc

  pltpu.sync_copy(out_v, out_hbm.at[pl.ds(base, B_PER_W)])


@jax.jit
def _mf(user_id, item_id, avg_score, user_embed, item_embed, ubias_flat,
        ibias_flat):
  mesh = plsc.VectorSubcoreMesh(core_axis_name="c", subcore_axis_name="s")
  cp = pltpu.CompilerParams()
  for field, val in (("needs_layout_passes", False),
                     ("use_tc_tiling_on_sc", False)):
    if field in pltpu.CompilerParams.__dataclass_fields__:
      cp = dataclasses.replace(cp, **{field: val})
  run = functools.partial(
      pl.kernel,
      compiler_params=cp,
      out_type=jax.ShapeDtypeStruct((BATCH,), jnp.float32),
      mesh=mesh,
      scratch_types=[
          pltpu.VMEM((N_CHUNKS, IDX_CHUNK), jnp.int32),
          pltpu.VMEM((N_CHUNKS, IDX_CHUNK), jnp.int32),
          pltpu.VMEM((B_PER_W,), jnp.float32),
          pltpu.VMEM((HIDDEN * N_CHUNKS, IDX_CHUNK), jnp.float32),
          pltpu.VMEM((HIDDEN * N_CHUNKS, IDX_CHUNK), jnp.float32),
          pltpu.VMEM((B_PER_W,), jnp.float32),
          pltpu.VMEM((B_PER_W,), jnp.float32),
          pltpu.VMEM((B_PER_W,), jnp.float32),
          pltpu.SemaphoreType.DMA,
          pltpu.SemaphoreType.DMA,
          pltpu.SemaphoreType.DMA,
          pltpu.SemaphoreType.DMA,
      ],
  )(_mf_kernel)
  return run(user_id, item_id, avg_score, user_embed.T, item_embed.T,
             ubias_flat, ibias_flat)


def kernel(user_id, item_id, avg_score, user_embed, item_embed, user_bias,
           item_bias):
  return _mf(user_id.astype(jnp.int32), item_id.astype(jnp.int32), avg_score,
             user_embed, item_embed, user_bias.reshape(-1),
             item_bias.reshape(-1))


# speedup vs baseline: 4.4393x; 4.4393x over previous
import dataclasses
import functools

import jax
import jax.numpy as jnp
from jax import lax
from jax.experimental import pallas as pl
from jax.experimental.pallas import tpu as pltpu
from jax.experimental.pallas import tpu_sc as plsc

BATCH = 16384
HIDDEN = 32
NUM_CORES = 2
NUM_SUBCORES = 16
NUM_WORKERS = NUM_CORES * NUM_SUBCORES
B_PER_W = BATCH // NUM_WORKERS
LANES = 16
GROUPS = B_PER_W // LANES
IDX_CHUNK = 128
N_CHUNKS = B_PER_W // IDX_CHUNK


def _mf_kernel(uid_hbm, iid_hbm, avg_hbm, uemb_hbm, iemb_hbm, ubias_hbm,
               ibias_hbm, out_hbm, uid_v, iid_v, avg_v, rows_u, rows_i,
               ub_v, ib_v, out_v, sem0, sem1, sem2, sem3):
  wid = lax.axis_index("s") * NUM_CORES + lax.axis_index("c")
  base = wid * B_PER_W

  for k in range(N_CHUNKS):
    pltpu.sync_copy(uid_hbm.at[pl.ds(base + k * IDX_CHUNK, IDX_CHUNK)],
                    uid_v.at[k])
    pltpu.sync_copy(iid_hbm.at[pl.ds(base + k * IDX_CHUNK, IDX_CHUNK)],
                    iid_v.at[k])
  pltpu.sync_copy(avg_hbm.at[pl.ds(base, B_PER_W)], avg_v)

  copies = []
  for k in range(N_CHUNKS):
    lo = k * IDX_CHUNK
    copies.append(pltpu.async_copy(
        uemb_hbm.at[uid_v.at[k]], rows_u.at[pl.ds(lo, IDX_CHUNK), :], sem0))
    copies.append(pltpu.async_copy(
        iemb_hbm.at[iid_v.at[k]], rows_i.at[pl.ds(lo, IDX_CHUNK), :], sem1))
    copies.append(pltpu.async_copy(
        ubias_hbm.at[uid_v.at[k]], ub_v.at[pl.ds(lo, IDX_CHUNK)], sem2))
    copies.append(pltpu.async_copy(
        ibias_hbm.at[iid_v.at[k]], ib_v.at[pl.ds(lo, IDX_CHUNK)], sem3))
  for c in copies:
    c.wait()

  lane_iota = lax.iota(jnp.int32, LANES)

  @pl.loop(0, GROUPS)
  def _(g):
    b0 = g * LANES
    bias = avg_v[pl.ds(b0, LANES)] + ub_v[pl.ds(b0, LANES)] + ib_v[pl.ds(b0, LANES)]
    acc = jnp.zeros((LANES,), jnp.float32)
    for j in range(LANES):
      b = b0 + j
      u0 = rows_u[b, pl.ds(0, LANES)]
      u1 = rows_u[b, pl.ds(LANES, LANES)]
      v0 = rows_i[b, pl.ds(0, LANES)]
      v1 = rows_i[b, pl.ds(LANES, LANES)]
      s = jnp.sum(u0 * v0 + u1 * v1)
      acc = jnp.where(lane_iota == j, s, acc)
    out_v[pl.ds(b0, LANES)] = acc + bias

  pltpu.sync_copy(out_v, out_hbm.at[pl.ds(base, B_PER_W)])


@jax.jit
def _mf(user_id, item_id, avg_score, user_embed, item_embed, ubias_flat,
        ibias_flat):
  mesh = plsc.VectorSubcoreMesh(core_axis_name="c", subcore_axis_name="s")
  cp = pltpu.CompilerParams()
  for field, val in (("needs_layout_passes", False),
                     ("use_tc_tiling_on_sc", False)):
    if field in pltpu.CompilerParams.__dataclass_fields__:
      cp = dataclasses.replace(cp, **{field: val})
  run = functools.partial(
      pl.kernel,
      compiler_params=cp,
      out_type=jax.ShapeDtypeStruct((BATCH,), jnp.float32),
      mesh=mesh,
      scratch_types=[
          pltpu.VMEM((N_CHUNKS, IDX_CHUNK), jnp.int32),
          pltpu.VMEM((N_CHUNKS, IDX_CHUNK), jnp.int32),
          pltpu.VMEM((B_PER_W,), jnp.float32),
          pltpu.VMEM((B_PER_W, HIDDEN), jnp.float32),
          pltpu.VMEM((B_PER_W, HIDDEN), jnp.float32),
          pltpu.VMEM((B_PER_W,), jnp.float32),
          pltpu.VMEM((B_PER_W,), jnp.float32),
          pltpu.VMEM((B_PER_W,), jnp.float32),
          pltpu.SemaphoreType.DMA,
          pltpu.SemaphoreType.DMA,
          pltpu.SemaphoreType.DMA,
          pltpu.SemaphoreType.DMA,
      ],
  )(_mf_kernel)
  return run(user_id, item_id, avg_score, user_embed, item_embed, ubias_flat,
             ibias_flat)


def kernel(user_id, item_id, avg_score, user_embed, item_embed, user_bias,
           item_bias):
  eye = lax.optimization_barrier(jnp.eye(HIDDEN, dtype=jnp.float32))
  return _mf(user_id.astype(jnp.int32), item_id.astype(jnp.int32), avg_score,
             user_embed @ eye, item_embed @ eye, user_bias.reshape(-1),
             item_bias.reshape(-1))


# speedup vs baseline: 4.7630x; 1.0729x over previous
import dataclasses
import functools

import jax
import jax.numpy as jnp
from jax import lax
from jax.experimental import pallas as pl
from jax.experimental.pallas import tpu as pltpu
from jax.experimental.pallas import tpu_sc as plsc

BATCH = 16384
HIDDEN = 32
NUM_CORES = 2
NUM_SUBCORES = 16
NUM_WORKERS = NUM_CORES * NUM_SUBCORES
B_PER_W = BATCH // NUM_WORKERS
LANES = 16
GROUPS = B_PER_W // LANES
IDX_CHUNK = 128
N_CHUNKS = B_PER_W // IDX_CHUNK


def _mf_kernel(uid_hbm, iid_hbm, avg_hbm, uemb_hbm, iemb_hbm, ubias_hbm,
               ibias_hbm, out_hbm, uid_v, iid_v, avg_v, rows_u, rows_i,
               ub_v, ib_v, out_v, sem0, sem1, sem2, sem3):
  wid = lax.axis_index("s") * NUM_CORES + lax.axis_index("c")
  base = wid * B_PER_W

  for k in range(N_CHUNKS):
    pltpu.sync_copy(uid_hbm.at[pl.ds(base + k * IDX_CHUNK, IDX_CHUNK)],
                    uid_v.at[k])
    pltpu.sync_copy(iid_hbm.at[pl.ds(base + k * IDX_CHUNK, IDX_CHUNK)],
                    iid_v.at[k])
  pltpu.sync_copy(avg_hbm.at[pl.ds(base, B_PER_W)], avg_v)

  copies = []
  for k in range(N_CHUNKS):
    lo = k * IDX_CHUNK
    copies.append(pltpu.async_copy(
        uemb_hbm.at[uid_v.at[k]], rows_u.at[pl.ds(lo, IDX_CHUNK), :], sem0))
    copies.append(pltpu.async_copy(
        iemb_hbm.at[iid_v.at[k]], rows_i.at[pl.ds(lo, IDX_CHUNK), :], sem1))
    copies.append(pltpu.async_copy(
        ubias_hbm.at[uid_v.at[k]], ub_v.at[pl.ds(lo, IDX_CHUNK)], sem2))
    copies.append(pltpu.async_copy(
        ibias_hbm.at[iid_v.at[k]], ib_v.at[pl.ds(lo, IDX_CHUNK)], sem3))
  for c in copies:
    c.wait()

  lane_iota = lax.iota(jnp.int32, LANES)

  @pl.loop(0, GROUPS)
  def _(g):
    b0 = g * LANES
    bias = avg_v[pl.ds(b0, LANES)] + ub_v[pl.ds(b0, LANES)] + ib_v[pl.ds(b0, LANES)]
    acc = jnp.zeros((LANES,), jnp.float32)
    for j in range(LANES):
      b = b0 + j
      u0, u1 = plsc.unpack(rows_u[b, pl.ds(0, HIDDEN)],
                           format=plsc.PackFormat.INTERLEAVED)
      v0, v1 = plsc.unpack(rows_i[b, pl.ds(0, HIDDEN)],
                           format=plsc.PackFormat.INTERLEAVED)
      s = jnp.sum(u0 * v0 + u1 * v1)
      acc = jnp.where(lane_iota == j, s, acc)
    out_v[pl.ds(b0, LANES)] = acc + bias

  pltpu.sync_copy(out_v, out_hbm.at[pl.ds(base, B_PER_W)])


@jax.jit
def _mf(user_id, item_id, avg_score, user_embed, item_embed, ubias_flat,
        ibias_flat):
  mesh = plsc.VectorSubcoreMesh(core_axis_name="c", subcore_axis_name="s")
  cp = pltpu.CompilerParams()
  for field, val in (("needs_layout_passes", False),
                     ("use_tc_tiling_on_sc", False)):
    if field in pltpu.CompilerParams.__dataclass_fields__:
      cp = dataclasses.replace(cp, **{field: val})
  run = functools.partial(
      pl.kernel,
      compiler_params=cp,
      out_type=jax.ShapeDtypeStruct((BATCH,), jnp.float32),
      mesh=mesh,
      scratch_types=[
          pltpu.VMEM((N_CHUNKS, IDX_CHUNK), jnp.int32),
          pltpu.VMEM((N_CHUNKS, IDX_CHUNK), jnp.int32),
          pltpu.VMEM((B_PER_W,), jnp.float32),
          pltpu.VMEM((B_PER_W, HIDDEN), jnp.bfloat16),
          pltpu.VMEM((B_PER_W, HIDDEN), jnp.bfloat16),
          pltpu.VMEM((B_PER_W,), jnp.float32),
          pltpu.VMEM((B_PER_W,), jnp.float32),
          pltpu.VMEM((B_PER_W,), jnp.float32),
          pltpu.SemaphoreType.DMA,
          pltpu.SemaphoreType.DMA,
          pltpu.SemaphoreType.DMA,
          pltpu.SemaphoreType.DMA,
      ],
  )(_mf_kernel)
  return run(user_id, item_id, avg_score, user_embed, item_embed, ubias_flat,
             ibias_flat)


def kernel(user_id, item_id, avg_score, user_embed, item_embed, user_bias,
           item_bias):
  return _mf(user_id.astype(jnp.int32), item_id.astype(jnp.int32), avg_score,
             user_embed.astype(jnp.bfloat16), item_embed.astype(jnp.bfloat16),
             user_bias.reshape(-1), item_bias.reshape(-1))


# speedup vs baseline: 5.8057x; 1.2189x over previous
import dataclasses
import functools

import jax
import jax.numpy as jnp
from jax import lax
from jax.experimental import pallas as pl
from jax.experimental.pallas import tpu as pltpu
from jax.experimental.pallas import tpu_sc as plsc

BATCH = 16384
HIDDEN = 32
NUM_CORES = 2
NUM_SUBCORES = 16
NUM_WORKERS = NUM_CORES * NUM_SUBCORES
B_PER_W = BATCH // NUM_WORKERS
LANES = 16
GROUPS = B_PER_W // LANES
IDX_CHUNK = 128
N_CHUNKS = B_PER_W // IDX_CHUNK


def _mf_kernel(uid_hbm, iid_hbm, avg_hbm, uemb_hbm, iemb_hbm, ubias_hbm,
               ibias_hbm, out_hbm, uid_v, iid_v, avg_v, rows_u, rows_i,
               ub_v, ib_v, out_v, sem0, sem1, sem2, sem3):
  wid = lax.axis_index("s") * NUM_CORES + lax.axis_index("c")
  base = wid * B_PER_W

  for k in range(N_CHUNKS):
    pltpu.sync_copy(uid_hbm.at[pl.ds(base + k * IDX_CHUNK, IDX_CHUNK)],
                    uid_v.at[k])
    pltpu.sync_copy(iid_hbm.at[pl.ds(base + k * IDX_CHUNK, IDX_CHUNK)],
                    iid_v.at[k])
  pltpu.sync_copy(avg_hbm.at[pl.ds(base, B_PER_W)], avg_v)

  copies = []
  for k in range(N_CHUNKS):
    lo = k * IDX_CHUNK
    copies.append(pltpu.async_copy(
        uemb_hbm.at[uid_v.at[k]], rows_u.at[pl.ds(lo, IDX_CHUNK), :], sem0))
    copies.append(pltpu.async_copy(
        iemb_hbm.at[iid_v.at[k]], rows_i.at[pl.ds(lo, IDX_CHUNK), :], sem1))
    copies.append(pltpu.async_copy(
        ubias_hbm.at[uid_v.at[k]], ub_v.at[pl.ds(lo, IDX_CHUNK)], sem2))
    copies.append(pltpu.async_copy(
        ibias_hbm.at[iid_v.at[k]], ib_v.at[pl.ds(lo, IDX_CHUNK)], sem3))
  for c in copies:
    c.wait()

  lane_iota = lax.iota(jnp.int32, LANES)

  @pl.loop(0, GROUPS)
  def _(g):
    b0 = g * LANES
    bias = avg_v[pl.ds(b0, LANES)] + ub_v[pl.ds(b0, LANES)] + ib_v[pl.ds(b0, LANES)]
    acc = jnp.zeros((LANES,), jnp.float32)
    for j in range(LANES):
      b = b0 + j
      u0 = rows_u[b, pl.ds(0, LANES)]
      u1 = rows_u[b, pl.ds(LANES, LANES)]
      v0 = rows_i[b, pl.ds(0, LANES)]
      v1 = rows_i[b, pl.ds(LANES, LANES)]
      s = jnp.sum(u0 * v0 + u1 * v1)
      acc = jnp.where(lane_iota == j, s, acc)
    out_v[pl.ds(b0, LANES)] = acc + bias

  pltpu.sync_copy(out_v, out_hbm.at[pl.ds(base, B_PER_W)])


@jax.jit
def _mf(user_id, item_id, avg_score, user_embed, item_embed, ubias_flat,
        ibias_flat):
  mesh = plsc.VectorSubcoreMesh(core_axis_name="c", subcore_axis_name="s")
  cp = pltpu.CompilerParams()
  for field, val in (("needs_layout_passes", False),
                     ("use_tc_tiling_on_sc", False)):
    if field in pltpu.CompilerParams.__dataclass_fields__:
      cp = dataclasses.replace(cp, **{field: val})
  run = functools.partial(
      pl.kernel,
      compiler_params=cp,
      out_type=jax.ShapeDtypeStruct((BATCH,), jnp.float32),
      mesh=mesh,
      scratch_types=[
          pltpu.VMEM((N_CHUNKS, IDX_CHUNK), jnp.int32),
          pltpu.VMEM((N_CHUNKS, IDX_CHUNK), jnp.int32),
          pltpu.VMEM((B_PER_W,), jnp.float32),
          pltpu.VMEM((B_PER_W, HIDDEN), jnp.float32),
          pltpu.VMEM((B_PER_W, HIDDEN), jnp.float32),
          pltpu.VMEM((B_PER_W,), jnp.float32),
          pltpu.VMEM((B_PER_W,), jnp.float32),
          pltpu.VMEM((B_PER_W,), jnp.float32),
          pltpu.SemaphoreType.DMA,
          pltpu.SemaphoreType.DMA,
          pltpu.SemaphoreType.DMA,
          pltpu.SemaphoreType.DMA,
      ],
  )(_mf_kernel)
  return run(user_id, item_id, avg_score, user_embed, item_embed, ubias_flat,
             ibias_flat)


def kernel(user_id, item_id, avg_score, user_embed, item_embed, user_bias,
           item_bias):
  return _mf(user_id.astype(jnp.int32), item_id.astype(jnp.int32), avg_score,
             user_embed, item_embed, user_bias.reshape(-1),
             item_bias.reshape(-1))
